# baseline (speedup 1.0000x reference)
import functools

import jax
import jax.numpy as jnp
from jax import lax
from jax.experimental import pallas as pl
from jax.experimental.pallas import tpu as pltpu
from jax.experimental.pallas import tpu_sc as plsc

_B = 1024
_F = 100
_DENSITY = 10000
_D = 32
_NW = 32
_NUNITS = _F * 4
_UPW = 13
_SLAB = 10240
_CS_MAX = 989696
_TAIL0 = 999936
_TAILN = 64


def _sc_body(xt_hbm, tablet_hbm, tail_hbm, out_hbm,
             xbuf, xbuf2, m_v, slab, stage, tail_v,
             sem_s, sem_x, sem_w0, sem_w1):
    nc = 2
    wid = lax.axis_index("s") * nc + lax.axis_index("c")

    pltpu.sync_copy(tail_hbm, tail_v)

    def _start_slab(i, g):
        cs = jnp.minimum((i * _DENSITY) // 128 * 128, _CS_MAX)
        pad0 = i * _DENSITY - cs
        h = pltpu.async_copy(
            tablet_hbm.at[pl.ds(g * 8, 8), pl.ds(cs, _SLAB)], slab, sem_s)
        return h, pad0

    def _m_loop(xb, i):
        irow = i % 8

        @plsc.parallel_loop(0, _B // 16, 1, unroll=2)
        def m_body(v_i):
            v = xb[irow, pl.ds(v_i * 16, 16)]
            tr = v.astype(jnp.int32)
            fl = jnp.where(v < tr.astype(jnp.float32), tr - 1, tr)
            m0 = lax.rem(fl, jnp.int32(_DENSITY))
            m0 = jnp.where(m0 < 0, m0 + _DENSITY, m0)
            m_v[pl.ds(v_i * 16, 16)] = m0

    def _gather_passes(i, g, pad0, tail_path):
        tloc = _TAIL0 - (_F - 1) * _DENSITY
        wb = {}
        for k in range(4):
            if k >= 2:
                wb[k - 2].wait()

            @plsc.parallel_loop(0, _B // 16, 1, unroll=3)
            def x_body(v_i, _k=k):
                mk = m_v[pl.ds(v_i * 16, 16)] + _k
                mk = jnp.where(mk >= _DENSITY, mk - _DENSITY, mk)
                if not tail_path:
                    j = mk + pad0
                    for c in range(8):
                        c_idx = jnp.full((16,), c, jnp.int32)
                        val = plsc.load_gather(slab, [c_idx, j])
                        stage[_k % 2, c, pl.ds(v_i * 16, 16)] = val
                else:
                    in_tail = mk >= tloc
                    j = jnp.where(in_tail, 0, mk + pad0)
                    jt = jnp.where(in_tail, mk - tloc, 0)
                    for c in range(8):
                        c_idx = jnp.full((16,), c, jnp.int32)
                        val = plsc.load_gather(slab, [c_idx, j])
                        tv = plsc.load_gather(
                            tail_v, [(g * 8 + c) * _TAILN + jt])
                        stage[_k % 2, c, pl.ds(v_i * 16, 16)] = jnp.where(
                            in_tail, tv, val)

            wb[k] = pltpu.async_copy(
                stage.at[k % 2], out_hbm.at[i, k, pl.ds(g * 8, 8), :],
                sem_w0 if k % 2 == 0 else sem_w1)
        return wb

    def _dslot(s, carry):
        uA = wid + (2 * s) * _NW
        uB = uA + _NW
        iA, gA = uA // 4, uA % 4
        iB, gB = uB // 4, uB % 4

        hA, padA = _start_slab(iA, gA)
        pltpu.sync_copy(xt_hbm.at[pl.ds((iA // 8) * 8, 8), :], xbuf)
        _m_loop(xbuf, iA)
        hx = pltpu.async_copy(
            xt_hbm.at[pl.ds((iB // 8) * 8, 8), :], xbuf2, sem_x)
        hA.wait()
        wbA = _gather_passes(iA, gA, padA, False)
        hB, padB = _start_slab(iB, gB)
        wbA[2].wait()
        wbA[3].wait()
        hx.wait()
        _m_loop(xbuf2, iB)
        hB.wait()
        wbB = _gather_passes(iB, gB, padB, False)
        wbB[2].wait()
        wbB[3].wait()
        return carry

    lax.fori_loop(0, (_UPW - 1) // 2, _dslot, 0)

    u = wid + (_UPW - 1) * _NW

    @pl.when(u < _NUNITS)
    def _last():
        i = u // 4
        g = u % 4
        h_slab, pad0 = _start_slab(i, g)
        pltpu.sync_copy(xt_hbm.at[pl.ds((i // 8) * 8, 8), :], xbuf)
        _m_loop(xbuf, i)
        h_slab.wait()

        @pl.when(i < _F - 1)
        def _main():
            wb = _gather_passes(i, g, pad0, False)
            wb[2].wait()
            wb[3].wait()

        @pl.when(i == _F - 1)
        def _tail():
            wb = _gather_passes(i, g, pad0, True)
            wb[2].wait()
            wb[3].wait()


@jax.jit
def _coeff_gather(xt, tablet, tail):
    mesh = plsc.VectorSubcoreMesh(
        core_axis_name="c", subcore_axis_name="s", num_cores=2,
        num_subcores=16,
    )
    f = pl.kernel(
        _sc_body,
        out_type=jax.ShapeDtypeStruct((_F, 4, _D, _B), jnp.float32),
        mesh=mesh,
        compiler_params=pltpu.CompilerParams(
            needs_layout_passes=False, use_tc_tiling_on_sc=True),
        scratch_types=[
            pltpu.VMEM((8, _B), jnp.float32),
            pltpu.VMEM((8, _B), jnp.float32),
            pltpu.VMEM((_B,), jnp.int32),
            pltpu.VMEM((8, _SLAB), jnp.float32),
            pltpu.VMEM((2, 8, _B), jnp.float32),
            pltpu.VMEM((_D * _TAILN,), jnp.float32),
            pltpu.SemaphoreType.DMA,
            pltpu.SemaphoreType.DMA,
            pltpu.SemaphoreType.DMA,
            pltpu.SemaphoreType.DMA,
        ],
    )
    return f(xt, tablet, tail)


def kernel(inputs, table):
    tail = table.T[:, _TAIL0:].reshape(-1)
    out_t = _coeff_gather(inputs.T, table.T, tail)
    return out_t.transpose(3, 0, 1, 2)


# speedup vs baseline: 1.0551x; 1.0551x over previous
import functools

import jax
import jax.numpy as jnp
from jax import lax
from jax.experimental import pallas as pl
from jax.experimental.pallas import tpu as pltpu
from jax.experimental.pallas import tpu_sc as plsc

_B = 1024
_F = 100
_DENSITY = 10000
_D = 32
_NW = 32
_NUNITS = _F * 4
_UPW = 13
_SLAB = 10240
_CS_MAX = 989696
_TAIL0 = 999936
_TAILN = 64


def _sc_body(xt_hbm, tablet_hbm, tail_hbm, out_hbm,
             xbuf, xbuf2, m_v, slab, stage, tail_v,
             sem_s, sem_x, sem_w0, sem_w1):
    nc = 2
    wid = lax.axis_index("s") * nc + lax.axis_index("c")

    pltpu.sync_copy(tail_hbm, tail_v)

    def _start_slab(i, g):
        cs = jnp.minimum((i * _DENSITY) // 128 * 128, _CS_MAX)
        pad0 = i * _DENSITY - cs
        h = pltpu.async_copy(
            tablet_hbm.at[pl.ds(g * 8, 8), pl.ds(cs, _SLAB)], slab, sem_s)
        return h, pad0

    def _m_loop(xb, i):
        irow = i % 8

        @plsc.parallel_loop(0, _B // 16, 1, unroll=2)
        def m_body(v_i):
            v = xb[irow, pl.ds(v_i * 16, 16)]
            tr = v.astype(jnp.int32)
            fl = jnp.where(v < tr.astype(jnp.float32), tr - 1, tr)
            m0 = lax.rem(fl, jnp.int32(_DENSITY))
            m0 = jnp.where(m0 < 0, m0 + _DENSITY, m0)
            m_v[pl.ds(v_i * 16, 16)] = m0

    def _gather_passes(i, g, pad0, tail_path):
        tloc = _TAIL0 - (_F - 1) * _DENSITY
        wb = {}
        for k in range(4):
            if k >= 2:
                wb[k - 2].wait()

            @plsc.parallel_loop(0, _B // 16, 1, unroll=2)
            def x_body(v_i, _k=k):
                mk = m_v[pl.ds(v_i * 16, 16)] + _k
                mk = jnp.where(mk >= _DENSITY, mk - _DENSITY, mk)
                if not tail_path:
                    j = mk + pad0
                    for c in range(8):
                        c_idx = jnp.full((16,), c, jnp.int32)
                        val = plsc.load_gather(slab, [c_idx, j])
                        stage[_k % 2, c, pl.ds(v_i * 16, 16)] = val
                else:
                    in_tail = mk >= tloc
                    j = jnp.where(in_tail, 0, mk + pad0)
                    jt = jnp.where(in_tail, mk - tloc, 0)
                    for c in range(8):
                        c_idx = jnp.full((16,), c, jnp.int32)
                        val = plsc.load_gather(slab, [c_idx, j])
                        tv = plsc.load_gather(
                            tail_v, [(g * 8 + c) * _TAILN + jt])
                        stage[_k % 2, c, pl.ds(v_i * 16, 16)] = jnp.where(
                            in_tail, tv, val)

            wb[k] = pltpu.async_copy(
                stage.at[k % 2], out_hbm.at[i, k, pl.ds(g * 8, 8), :],
                sem_w0 if k % 2 == 0 else sem_w1)
        return wb

    xbufs = (xbuf, xbuf2)

    def _qslot(s, carry):
        us = [wid + (4 * s + q) * _NW for q in range(4)]
        ig = [(u // 4, u % 4) for u in us]

        h0, pad = _start_slab(*ig[0])
        pltpu.sync_copy(xt_hbm.at[pl.ds((ig[0][0] // 8) * 8, 8), :], xbuf)
        _m_loop(xbuf, ig[0][0])
        hx = pltpu.async_copy(
            xt_hbm.at[pl.ds((ig[1][0] // 8) * 8, 8), :], xbuf2, sem_x)
        h0.wait()
        wb = _gather_passes(ig[0][0], ig[0][1], pad, False)
        for q in range(1, 4):
            iq, gq = ig[q]
            hq, pad = _start_slab(iq, gq)
            wb[2].wait()
            wb[3].wait()
            hx.wait()
            _m_loop(xbufs[q % 2], iq)
            if q < 3:
                hx = pltpu.async_copy(
                    xt_hbm.at[pl.ds((ig[q + 1][0] // 8) * 8, 8), :],
                    xbufs[(q + 1) % 2], sem_x)
            hq.wait()
            wb = _gather_passes(iq, gq, pad, False)
        wb[2].wait()
        wb[3].wait()
        return carry

    lax.fori_loop(0, (_UPW - 1) // 4, _qslot, 0)

    u = wid + (_UPW - 1) * _NW

    @pl.when(u < _NUNITS)
    def _last():
        i = u // 4
        g = u % 4
        h_slab, pad0 = _start_slab(i, g)
        pltpu.sync_copy(xt_hbm.at[pl.ds((i // 8) * 8, 8), :], xbuf)
        _m_loop(xbuf, i)
        h_slab.wait()

        @pl.when(i < _F - 1)
        def _main():
            wb = _gather_passes(i, g, pad0, False)
            wb[2].wait()
            wb[3].wait()

        @pl.when(i == _F - 1)
        def _tail():
            wb = _gather_passes(i, g, pad0, True)
            wb[2].wait()
            wb[3].wait()


@jax.jit
def _coeff_gather(xt, tablet, tail):
    mesh = plsc.VectorSubcoreMesh(
        core_axis_name="c", subcore_axis_name="s", num_cores=2,
        num_subcores=16,
    )
    f = pl.kernel(
        _sc_body,
        out_type=jax.ShapeDtypeStruct((_F, 4, _D, _B), jnp.float32),
        mesh=mesh,
        compiler_params=pltpu.CompilerParams(
            needs_layout_passes=False, use_tc_tiling_on_sc=True),
        scratch_types=[
            pltpu.VMEM((8, _B), jnp.float32),
            pltpu.VMEM((8, _B), jnp.float32),
            pltpu.VMEM((_B,), jnp.int32),
            pltpu.VMEM((8, _SLAB), jnp.float32),
            pltpu.VMEM((2, 8, _B), jnp.float32),
            pltpu.VMEM((_D * _TAILN,), jnp.float32),
            pltpu.SemaphoreType.DMA,
            pltpu.SemaphoreType.DMA,
            pltpu.SemaphoreType.DMA,
            pltpu.SemaphoreType.DMA,
        ],
    )
    return f(xt, tablet, tail)


def kernel(inputs, table):
    tail = table.T[:, _TAIL0:].reshape(-1)
    out_t = _coeff_gather(inputs.T, table.T, tail)
    return out_t.transpose(3, 0, 1, 2)
